# baseline (speedup 1.0000x reference)
import functools

import jax
import jax.numpy as jnp
from jax import lax
from jax.experimental import pallas as pl
from jax.experimental.pallas import tpu as pltpu
from jax.experimental.pallas import tpu_sc as plsc

_CSTR = 32


def _sc_histogram(idx, B, L):
    info = plsc.get_sparse_core_info()
    NW = info.num_cores * info.num_subcores
    RW = B // NW
    G = 16
    NG = RW // G
    mesh = plsc.VectorSubcoreMesh(core_axis_name="c", subcore_axis_name="s")

    @functools.partial(
        pl.kernel,
        out_type=jax.ShapeDtypeStruct((B, _CSTR), jnp.float32),
        mesh=mesh,
        scratch_types=[
            pltpu.VMEM((2, G, L), jnp.int32),
            pltpu.VMEM((2, G, _CSTR), jnp.float32),
            pltpu.SemaphoreType.DMA,
            pltpu.SemaphoreType.DMA,
            pltpu.SemaphoreType.DMA,
            pltpu.SemaphoreType.DMA,
        ],
        compiler_params=pltpu.CompilerParams(needs_layout_passes=False),
    )
    def k(idx_hbm, out_hbm, blk, cnt, in0, in1, ot0, ot1):
        wid = lax.axis_index("s") * info.num_cores + lax.axis_index("c")
        base = wid * RW
        lane = lax.iota(jnp.int32, 16)
        ones = jnp.ones((16,), jnp.float32)
        zeros = jnp.zeros((16,), jnp.float32)
        Lvec = jnp.full((16,), L, jnp.int32)
        in_sems = (in0, in1)
        out_sems = (ot0, ot1)

        def start_in(g):
            p = g % 2
            return pltpu.async_copy(
                idx_hbm.at[pl.ds(base + g * G, G), :], blk.at[p], in_sems[p]
            )

        in_dma = [None] * NG
        out_dma = [None] * NG
        in_dma[0] = start_in(0)
        for g in range(NG):
            p = g % 2
            if g + 1 < NG:
                in_dma[g + 1] = start_in(g + 1)
            if g >= 2:
                out_dma[g - 2].wait()
            cnt_p = cnt.at[p]
            for r in range(G):
                for c in range(_CSTR // 16):
                    cnt_p[r, pl.ds(c * 16, 16)] = zeros
            in_dma[g].wait()
            blk_p = blk.at[p]

            @plsc.parallel_loop(0, L, unroll=8, carry=lane)
            def _pos(l, col):
                v = plsc.load_gather(blk_p, [lane, col])
                plsc.addupdate_scatter(cnt_p, [lane, v], ones)
                col2 = col + 1
                return jnp.where(col2 == Lvec, 0, col2)

            out_dma[g] = pltpu.async_copy(
                cnt_p, out_hbm.at[pl.ds(base + g * G, G), :], out_sems[p]
            )
        for g in range(max(NG - 2, 0), NG):
            out_dma[g].wait()

    return k(idx)


def _tc_dense(counts, emb_pad, W, b2, L, B, blk0, prev=None):
    Bc = counts.shape[0]
    OUT = W.shape[0]
    E = W.shape[1]
    TB = 1024
    Lf = float(L)

    def body(*refs):
        c_ref, e_ref, w_ref, b_ref = refs[-5:-1]
        o_ref = refs[-1]
        c = c_ref[...]
        s = jnp.dot(c, e_ref[...], preferred_element_type=jnp.float32)
        lens = jnp.maximum(Lf - c[:, 0:1], 1.0)
        pooled = s / lens
        h = lax.dot_general(
            pooled, w_ref[...], (((1,), (1,)), ((), ())),
            preferred_element_type=jnp.float32,
        ) + b_ref[...]
        o_ref[...] = h * (1.0 / (1.0 + jnp.exp(-h)))

    in_specs = [
        pl.BlockSpec((TB, _CSTR), lambda i: (i, 0)),
        pl.BlockSpec((_CSTR, E), lambda i: (0, 0)),
        pl.BlockSpec((OUT, E), lambda i: (0, 0)),
        pl.BlockSpec((1, OUT), lambda i: (0, 0)),
    ]
    args = [counts, emb_pad, W, b2]
    aliases = {}
    if prev is not None:
        in_specs = [pl.BlockSpec(memory_space=pl.ANY)] + in_specs
        args = [prev] + args
        aliases = {0: 0}
    return pl.pallas_call(
        body,
        grid=(Bc // TB,),
        in_specs=in_specs,
        out_specs=pl.BlockSpec((TB, OUT), lambda i: (i + blk0, 0)),
        out_shape=jax.ShapeDtypeStruct((B, OUT), jnp.float32),
        input_output_aliases=aliases,
    )(*args)


def kernel(prot_indices, emb_table, W, b):
    B, L = prot_indices.shape
    V, E = emb_table.shape
    idx = prot_indices
    if idx.dtype != jnp.int32:
        idx = idx.astype(jnp.int32)
    emb_pad = jnp.zeros((_CSTR, E), jnp.float32).at[:V].set(emb_table)
    b2 = b.reshape(1, -1)
    H = B // 2
    c0 = _sc_histogram(idx[:H], H, L)
    c1 = _sc_histogram(idx[H:], H, L)
    out = _tc_dense(c0, emb_pad, W, b2, L, B, 0)
    out = _tc_dense(c1, emb_pad, W, b2, L, B, H // 1024, prev=out)
    return out


# speedup vs baseline: 1.0892x; 1.0892x over previous
import functools

import jax
import jax.numpy as jnp
from jax import lax
from jax.experimental import pallas as pl
from jax.experimental.pallas import tpu as pltpu
from jax.experimental.pallas import tpu_sc as plsc

_CSTR = 32


def _sc_histogram(idx, B, L):
    info = plsc.get_sparse_core_info()
    NW = info.num_cores * info.num_subcores
    RW = B // NW
    G = 16
    NG = RW // G
    mesh = plsc.VectorSubcoreMesh(core_axis_name="c", subcore_axis_name="s")

    @functools.partial(
        pl.kernel,
        out_type=jax.ShapeDtypeStruct((B, _CSTR), jnp.float32),
        mesh=mesh,
        scratch_types=[
            pltpu.VMEM((2, G, L), jnp.int32),
            pltpu.VMEM((2, G, _CSTR), jnp.float32),
            pltpu.SemaphoreType.DMA,
            pltpu.SemaphoreType.DMA,
            pltpu.SemaphoreType.DMA,
            pltpu.SemaphoreType.DMA,
        ],
        compiler_params=pltpu.CompilerParams(needs_layout_passes=False, use_tc_tiling_on_sc=True),
    )
    def k(idx_hbm, out_hbm, blk, cnt, in0, in1, ot0, ot1):
        wid = lax.axis_index("s") * info.num_cores + lax.axis_index("c")
        base = wid * RW
        lane = lax.iota(jnp.int32, 16)
        ones = jnp.ones((16,), jnp.float32)
        zeros = jnp.zeros((16,), jnp.float32)
        Lvec = jnp.full((16,), L, jnp.int32)
        in_sems = (in0, in1)
        out_sems = (ot0, ot1)

        def start_in(g):
            p = g % 2
            return pltpu.async_copy(
                idx_hbm.at[pl.ds(base + g * G, G), :], blk.at[p], in_sems[p]
            )

        in_dma = [None] * NG
        out_dma = [None] * NG
        in_dma[0] = start_in(0)
        for g in range(NG):
            p = g % 2
            if g + 1 < NG:
                in_dma[g + 1] = start_in(g + 1)
            if g >= 2:
                out_dma[g - 2].wait()
            cnt_p = cnt.at[p]
            for r in range(G):
                for c in range(_CSTR // 16):
                    cnt_p[r, pl.ds(c * 16, 16)] = zeros
            in_dma[g].wait()
            blk_p = blk.at[p]

            @plsc.parallel_loop(0, L, unroll=8, carry=lane)
            def _pos(l, col):
                v = plsc.load_gather(blk_p, [lane, col])
                plsc.addupdate_scatter(cnt_p, [lane, v], ones)
                col2 = col + 1
                return jnp.where(col2 == Lvec, 0, col2)

            out_dma[g] = pltpu.async_copy(
                cnt_p, out_hbm.at[pl.ds(base + g * G, G), :], out_sems[p]
            )
        for g in range(max(NG - 2, 0), NG):
            out_dma[g].wait()

    return k(idx)


def _tc_dense(counts, emb_pad, W, b2, L, B, blk0, prev=None):
    Bc = counts.shape[0]
    OUT = W.shape[0]
    E = W.shape[1]
    TB = 1024
    Lf = float(L)

    def body(*refs):
        c_ref, e_ref, w_ref, b_ref = refs[-5:-1]
        o_ref = refs[-1]
        c = c_ref[...]
        s = jnp.dot(c, e_ref[...], preferred_element_type=jnp.float32)
        lens = jnp.maximum(Lf - c[:, 0:1], 1.0)
        pooled = s / lens
        h = lax.dot_general(
            pooled, w_ref[...], (((1,), (1,)), ((), ())),
            preferred_element_type=jnp.float32,
        ) + b_ref[...]
        o_ref[...] = h * (1.0 / (1.0 + jnp.exp(-h)))

    in_specs = [
        pl.BlockSpec((TB, _CSTR), lambda i: (i, 0)),
        pl.BlockSpec((_CSTR, E), lambda i: (0, 0)),
        pl.BlockSpec((OUT, E), lambda i: (0, 0)),
        pl.BlockSpec((1, OUT), lambda i: (0, 0)),
    ]
    args = [counts, emb_pad, W, b2]
    aliases = {}
    if prev is not None:
        in_specs = [pl.BlockSpec(memory_space=pl.ANY)] + in_specs
        args = [prev] + args
        aliases = {0: 0}
    return pl.pallas_call(
        body,
        grid=(Bc // TB,),
        in_specs=in_specs,
        out_specs=pl.BlockSpec((TB, OUT), lambda i: (i + blk0, 0)),
        out_shape=jax.ShapeDtypeStruct((B, OUT), jnp.float32),
        input_output_aliases=aliases,
    )(*args)


def kernel(prot_indices, emb_table, W, b):
    B, L = prot_indices.shape
    V, E = emb_table.shape
    idx = prot_indices
    if idx.dtype != jnp.int32:
        idx = idx.astype(jnp.int32)
    emb_pad = jnp.zeros((_CSTR, E), jnp.float32).at[:V].set(emb_table)
    b2 = b.reshape(1, -1)
    counts = _sc_histogram(idx, B, L)
    return _tc_dense(counts, emb_pad, W, b2, L, B, 0)


# speedup vs baseline: 1.1312x; 1.0385x over previous
import functools

import jax
import jax.numpy as jnp
from jax import lax
from jax.experimental import pallas as pl
from jax.experimental.pallas import tpu as pltpu
from jax.experimental.pallas import tpu_sc as plsc

_CSTR = 32


def _sc_histogram(idx, B, L):
    info = plsc.get_sparse_core_info()
    NW = info.num_cores * info.num_subcores
    RW = B // NW
    G = 16
    NG = RW // G
    mesh = plsc.VectorSubcoreMesh(core_axis_name="c", subcore_axis_name="s")

    @functools.partial(
        pl.kernel,
        out_type=jax.ShapeDtypeStruct((B, _CSTR), jnp.float32),
        mesh=mesh,
        scratch_types=[
            pltpu.VMEM((2, G, L), jnp.int32),
            pltpu.VMEM((2, G, _CSTR), jnp.float32),
            pltpu.SemaphoreType.DMA,
            pltpu.SemaphoreType.DMA,
            pltpu.SemaphoreType.DMA,
            pltpu.SemaphoreType.DMA,
        ],
        compiler_params=pltpu.CompilerParams(needs_layout_passes=False),
    )
    def k(idx_hbm, out_hbm, blk, cnt, in0, in1, ot0, ot1):
        wid = lax.axis_index("s") * info.num_cores + lax.axis_index("c")
        base = wid * RW
        lane = lax.iota(jnp.int32, 16)
        ones = jnp.ones((16,), jnp.float32)
        zeros = jnp.zeros((16,), jnp.float32)
        Lvec = jnp.full((16,), L, jnp.int32)
        in_sems = (in0, in1)
        out_sems = (ot0, ot1)

        def start_in(g):
            p = g % 2
            return pltpu.async_copy(
                idx_hbm.at[pl.ds(base + g * G, G), :], blk.at[p], in_sems[p]
            )

        in_dma = [None] * NG
        out_dma = [None] * NG
        in_dma[0] = start_in(0)
        for g in range(NG):
            p = g % 2
            if g + 1 < NG:
                in_dma[g + 1] = start_in(g + 1)
            if g >= 2:
                out_dma[g - 2].wait()
            cnt_p = cnt.at[p]
            for r in range(G):
                for c in range(_CSTR // 16):
                    cnt_p[r, pl.ds(c * 16, 16)] = zeros
            in_dma[g].wait()
            blk_p = blk.at[p]

            @plsc.parallel_loop(0, L - 16, unroll=8, carry=lane)
            def _main(l, col):
                v = plsc.load_gather(blk_p, [lane, col])
                plsc.addupdate_scatter(cnt_p, [lane, v], ones)
                return col + 1

            @plsc.parallel_loop(0, 16, unroll=8, carry=_main)
            def _tail(l, col):
                v = plsc.load_gather(blk_p, [lane, col])
                plsc.addupdate_scatter(cnt_p, [lane, v], ones)
                col2 = col + 1
                return jnp.where(col2 == Lvec, 0, col2)

            out_dma[g] = pltpu.async_copy(
                cnt_p, out_hbm.at[pl.ds(base + g * G, G), :], out_sems[p]
            )
        for g in range(max(NG - 2, 0), NG):
            out_dma[g].wait()

    return k(idx)


def _tc_dense(counts, emb_pad, W, b2, L, B, blk0, prev=None):
    Bc = counts.shape[0]
    OUT = W.shape[0]
    E = W.shape[1]
    TB = 2048
    Lf = float(L)

    def body(*refs):
        c_ref, e_ref, w_ref, b_ref = refs[-5:-1]
        o_ref = refs[-1]
        c = c_ref[...]
        s = jnp.dot(c, e_ref[...], preferred_element_type=jnp.float32)
        lens = jnp.maximum(Lf - c[:, 0:1], 1.0)
        pooled = s / lens
        h = lax.dot_general(
            pooled, w_ref[...], (((1,), (1,)), ((), ())),
            preferred_element_type=jnp.float32,
        ) + b_ref[...]
        o_ref[...] = h * (1.0 / (1.0 + jnp.exp(-h)))

    in_specs = [
        pl.BlockSpec((TB, _CSTR), lambda i: (i, 0)),
        pl.BlockSpec((_CSTR, E), lambda i: (0, 0)),
        pl.BlockSpec((OUT, E), lambda i: (0, 0)),
        pl.BlockSpec((1, OUT), lambda i: (0, 0)),
    ]
    args = [counts, emb_pad, W, b2]
    aliases = {}
    if prev is not None:
        in_specs = [pl.BlockSpec(memory_space=pl.ANY)] + in_specs
        args = [prev] + args
        aliases = {0: 0}
    return pl.pallas_call(
        body,
        grid=(Bc // TB,),
        in_specs=in_specs,
        out_specs=pl.BlockSpec((TB, OUT), lambda i: (i + blk0, 0)),
        out_shape=jax.ShapeDtypeStruct((B, OUT), jnp.float32),
        input_output_aliases=aliases,
    )(*args)


def kernel(prot_indices, emb_table, W, b):
    B, L = prot_indices.shape
    V, E = emb_table.shape
    idx = prot_indices
    if idx.dtype != jnp.int32:
        idx = idx.astype(jnp.int32)
    emb_pad = jnp.zeros((_CSTR, E), jnp.float32).at[:V].set(emb_table)
    b2 = b.reshape(1, -1)
    counts = _sc_histogram(idx, B, L)
    return _tc_dense(counts, emb_pad, W, b2, L, B, 0)
